# Initial kernel scaffold; baseline (speedup 1.0000x reference)
import jax
import jax.numpy as jnp
from jax.experimental import pallas as pl


def kernel(context, device_category, start_time, end_time, host, W):
    raise NotImplementedError("write your pallas kernel here")


# speedup vs baseline: 1.1038x; 1.1038x over previous
import functools

import jax
import jax.numpy as jnp
from jax import lax
from jax.experimental import pallas as pl
from jax.experimental.pallas import tpu as pltpu
from jax.experimental.pallas import tpu_sc as plsc

ALEN = 40
BATCH = 4096
EMB = 128
RB = 256
NBLK = BATCH // RB
_BIGI = 1 << 30


def _fused_body(ctx_ref, w_ref, catc_ref, catr_ref,
                eact_ref, eap_ref, idx_ref,
                e_sc, sqr_sc):
    i = pl.program_id(0)

    @pl.when(i == 0)
    def _init():
        e = lax.dot_general(ctx_ref[:, :ALEN], w_ref[...],
                            (((1,), (0,)), ((), ())),
                            preferred_element_type=jnp.float32)
        e_sc[...] = e
        sqr_sc[...] = jnp.sum(e * e, axis=1)[None, :]

    rows = pl.ds(i * RB, RB)
    e_rows = e_sc[rows, :]
    eact_ref[...] = e_rows
    eap_ref[...] = lax.dot_general(ctx_ref[rows, ALEN:], w_ref[...],
                                   (((1,), (0,)), ((), ())),
                                   preferred_element_type=jnp.float32)

    sq_rows = jnp.sum(e_rows * e_rows, axis=1, keepdims=True)
    g = lax.dot_general(e_rows, e_sc[...],
                        (((1,), (1,)), ((), ())),
                        preferred_element_type=jnp.float32)
    scores = (sq_rows + sqr_sc[...]) - 2.0 * g

    same = catc_ref[rows, :] == catr_ref[...]
    s = jnp.where(same, jnp.inf, scores)
    m = jnp.min(s, axis=1, keepdims=True)
    col = lax.broadcasted_iota(jnp.int32, (RB, BATCH), 1)
    idx = jnp.min(jnp.where(s == m, col, _BIGI), axis=1, keepdims=True)
    idx_ref[...] = jnp.broadcast_to(idx, (RB, 128))


_fused = pl.pallas_call(
    _fused_body,
    grid=(NBLK,),
    in_specs=[
        pl.BlockSpec((BATCH, 2 * ALEN), lambda i: (0, 0)),
        pl.BlockSpec((ALEN, EMB), lambda i: (0, 0)),
        pl.BlockSpec((BATCH, 1), lambda i: (0, 0)),
        pl.BlockSpec((1, BATCH), lambda i: (0, 0)),
    ],
    out_specs=[
        pl.BlockSpec((RB, EMB), lambda i: (i, 0)),
        pl.BlockSpec((RB, EMB), lambda i: (i, 0)),
        pl.BlockSpec((RB, 128), lambda i: (i, 0)),
    ],
    out_shape=[
        jax.ShapeDtypeStruct((BATCH, EMB), jnp.float32),
        jax.ShapeDtypeStruct((BATCH, EMB), jnp.float32),
        jax.ShapeDtypeStruct((BATCH, 128), jnp.int32),
    ],
    scratch_shapes=[
        pltpu.VMEM((BATCH, EMB), jnp.float32),
        pltpu.VMEM((1, BATCH), jnp.float32),
    ],
)


def _sc_gather(table, idx):
    info = plsc.get_sparse_core_info()
    nw = info.num_cores * info.num_subcores
    bpw = BATCH // nw
    mesh = plsc.VectorSubcoreMesh(core_axis_name="c", subcore_axis_name="s")

    @functools.partial(
        pl.kernel, mesh=mesh,
        out_type=jax.ShapeDtypeStruct((BATCH, EMB), jnp.float32),
        scratch_types=[
            pltpu.VMEM((bpw,), jnp.int32),
            pltpu.VMEM((bpw, EMB), jnp.float32),
            pltpu.SemaphoreType.DMA,
        ],
    )
    def gather(table_hbm, idx_hbm, out_hbm, idx_v, rows_v, sem):
        wid = lax.axis_index("s") * info.num_cores + lax.axis_index("c")
        base = wid * bpw
        pltpu.sync_copy(idx_hbm.at[pl.ds(base, bpw)], idx_v)
        pltpu.async_copy(table_hbm.at[idx_v], rows_v, sem).wait()
        pltpu.sync_copy(rows_v, out_hbm.at[pl.ds(base, bpw)])

    return gather(table, idx)


def kernel(context, device_category, start_time, end_time, host, W):
    cat = device_category.astype(jnp.int32)
    e_actv, e_ap, idx2d = _fused(context, W, cat[:, None], cat[None, :])
    e_an = _sc_gather(e_actv, idx2d[:, 0])
    return (e_actv, e_ap, e_an)


# speedup vs baseline: 1.2744x; 1.1546x over previous
import functools

import jax
import jax.numpy as jnp
from jax import lax
from jax.experimental import pallas as pl
from jax.experimental.pallas import tpu as pltpu
from jax.experimental.pallas import tpu_sc as plsc

ALEN = 40
BATCH = 4096
EMB = 128
NCAT = 64
AUGW = EMB + NCAT + 3
RB = 256
NBLK = BATCH // RB
_BIGI = 1 << 30
_ALPHA = float(2.0 ** 25)


def _fused_body(ctx_ref, w_ref, catc_ref, eact_ref, eap_ref, idx_ref,
                aug_sc, aug2_sc):
    i = pl.program_id(0)

    @pl.when(i == 0)
    def _init():
        e = lax.dot_general(ctx_ref[:, :ALEN], w_ref[...],
                            (((1,), (0,)), ((), ())),
                            preferred_element_type=jnp.float32)
        lane = lax.broadcasted_iota(jnp.int32, (BATCH, NCAT), 1)
        oh = jnp.where(catc_ref[...] == lane, _ALPHA, 0.0)
        sq = jnp.sum(e * e, axis=1, keepdims=True)
        s1 = sq.astype(jnp.bfloat16).astype(jnp.float32)
        r1 = sq - s1
        s2 = r1.astype(jnp.bfloat16).astype(jnp.float32)
        s3 = r1 - s2
        one = jnp.ones((BATCH, 1), jnp.float32)
        aug_sc[...] = jnp.concatenate([e, oh, s1, s2, s3], axis=1)
        aug2_sc[...] = jnp.concatenate([-2.0 * e, oh, one, one, one], axis=1)

    cols = pl.ds(i * RB, RB)
    eact_ref[...] = aug_sc[cols, 0:EMB]
    eap_ref[...] = lax.dot_general(ctx_ref[cols, ALEN:], w_ref[...],
                                   (((1,), (0,)), ((), ())),
                                   preferred_element_type=jnp.float32)

    t = lax.dot_general(aug_sc[...], aug2_sc[cols, :],
                        (((1,), (1,)), ((), ())),
                        preferred_element_type=jnp.float32)
    m = jnp.min(t, axis=0, keepdims=True)
    row = lax.broadcasted_iota(jnp.int32, (BATCH, RB), 0)
    idx = jnp.min(jnp.where(t == m, row, _BIGI), axis=0, keepdims=True)
    idx_ref[...] = idx.reshape(1, 1, RB)


_fused = pl.pallas_call(
    _fused_body,
    grid=(NBLK,),
    in_specs=[
        pl.BlockSpec((BATCH, 2 * ALEN), lambda i: (0, 0)),
        pl.BlockSpec((ALEN, EMB), lambda i: (0, 0)),
        pl.BlockSpec((BATCH, 1), lambda i: (0, 0)),
    ],
    out_specs=[
        pl.BlockSpec((RB, EMB), lambda i: (i, 0)),
        pl.BlockSpec((RB, EMB), lambda i: (i, 0)),
        pl.BlockSpec((1, 1, RB), lambda i: (i, 0, 0)),
    ],
    out_shape=[
        jax.ShapeDtypeStruct((BATCH, EMB), jnp.float32),
        jax.ShapeDtypeStruct((BATCH, EMB), jnp.float32),
        jax.ShapeDtypeStruct((NBLK, 1, RB), jnp.int32),
    ],
    scratch_shapes=[
        pltpu.VMEM((BATCH, AUGW), jnp.float32),
        pltpu.VMEM((BATCH, AUGW), jnp.float32),
    ],
)


def _sc_gather(table, idx):
    info = plsc.get_sparse_core_info()
    nw = info.num_cores * info.num_subcores
    bpw = BATCH // nw
    mesh = plsc.VectorSubcoreMesh(core_axis_name="c", subcore_axis_name="s")

    @functools.partial(
        pl.kernel, mesh=mesh,
        out_type=jax.ShapeDtypeStruct((BATCH, EMB), jnp.float32),
        scratch_types=[
            pltpu.VMEM((bpw,), jnp.int32),
            pltpu.VMEM((bpw, EMB), jnp.float32),
            pltpu.SemaphoreType.DMA,
        ],
    )
    def gather(table_hbm, idx_hbm, out_hbm, idx_v, rows_v, sem):
        wid = lax.axis_index("s") * info.num_cores + lax.axis_index("c")
        base = wid * bpw
        pltpu.sync_copy(idx_hbm.at[pl.ds(base, bpw)], idx_v)
        pltpu.async_copy(table_hbm.at[idx_v], rows_v, sem).wait()
        pltpu.sync_copy(rows_v, out_hbm.at[pl.ds(base, bpw)])

    return gather(table, idx)


def kernel(context, device_category, start_time, end_time, host, W):
    cat = device_category.astype(jnp.int32)
    e_actv, e_ap, idx3d = _fused(context, W, cat[:, None])
    e_an = _sc_gather(e_actv, idx3d.reshape(BATCH))
    return (e_actv, e_ap, e_an)


# speedup vs baseline: 1.2923x; 1.0140x over previous
import functools

import jax
import jax.numpy as jnp
from jax import lax
from jax.experimental import pallas as pl
from jax.experimental.pallas import tpu as pltpu
from jax.experimental.pallas import tpu_sc as plsc

ALEN = 40
BATCH = 4096
EMB = 128
NCAT = 64
AUGW = EMB + NCAT + 3
RB = 256
NBLK = BATCH // RB
_BIGI = 1 << 30
_ALPHA = float(2.0 ** 25)


def _fused_body(ctx_ref, w_ref, catc_ref, eact_ref, eap_ref, idx_ref,
                e_sc, aug_sc, aug2_sc):
    i = pl.program_id(0)

    @pl.when(i == 0)
    def _init():
        e = lax.dot_general(ctx_ref[:, :ALEN], w_ref[...],
                            (((1,), (0,)), ((), ())),
                            preferred_element_type=jnp.float32)
        e_sc[...] = e
        lane = lax.broadcasted_iota(jnp.int32, (BATCH, NCAT), 1)
        oh = jnp.where(catc_ref[...] == lane, _ALPHA, 0.0)
        sq = jnp.sum(e * e, axis=1, keepdims=True)
        s1 = sq.astype(jnp.bfloat16).astype(jnp.float32)
        r1 = sq - s1
        s2 = r1.astype(jnp.bfloat16).astype(jnp.float32)
        s3 = r1 - s2
        one = jnp.ones((BATCH, 1), jnp.float32)
        aug_sc[...] = jnp.concatenate(
            [e, oh, s1, s2, s3], axis=1).astype(jnp.bfloat16)
        aug2_sc[...] = jnp.concatenate(
            [-2.0 * e, oh, one, one, one], axis=1).astype(jnp.bfloat16)

    cols = pl.ds(i * RB, RB)
    eact_ref[...] = e_sc[cols, :]
    eap_ref[...] = lax.dot_general(ctx_ref[cols, ALEN:], w_ref[...],
                                   (((1,), (0,)), ((), ())),
                                   preferred_element_type=jnp.float32)

    t = lax.dot_general(aug_sc[...], aug2_sc[cols, :],
                        (((1,), (1,)), ((), ())),
                        preferred_element_type=jnp.float32)
    m = jnp.min(t, axis=0, keepdims=True)
    row = lax.broadcasted_iota(jnp.int32, (BATCH, RB), 0)
    idx = jnp.min(jnp.where(t == m, row, _BIGI), axis=0, keepdims=True)
    idx_ref[...] = idx.reshape(1, 1, RB)


_fused = pl.pallas_call(
    _fused_body,
    grid=(NBLK,),
    in_specs=[
        pl.BlockSpec((BATCH, 2 * ALEN), lambda i: (0, 0)),
        pl.BlockSpec((ALEN, EMB), lambda i: (0, 0)),
        pl.BlockSpec((BATCH, 1), lambda i: (0, 0)),
    ],
    out_specs=[
        pl.BlockSpec((RB, EMB), lambda i: (i, 0)),
        pl.BlockSpec((RB, EMB), lambda i: (i, 0)),
        pl.BlockSpec((1, 1, RB), lambda i: (i, 0, 0)),
    ],
    out_shape=[
        jax.ShapeDtypeStruct((BATCH, EMB), jnp.float32),
        jax.ShapeDtypeStruct((BATCH, EMB), jnp.float32),
        jax.ShapeDtypeStruct((NBLK, 1, RB), jnp.int32),
    ],
    scratch_shapes=[
        pltpu.VMEM((BATCH, EMB), jnp.float32),
        pltpu.VMEM((BATCH, AUGW), jnp.bfloat16),
        pltpu.VMEM((BATCH, AUGW), jnp.bfloat16),
    ],
)


def _sc_gather(table, idx):
    info = plsc.get_sparse_core_info()
    nw = info.num_cores * info.num_subcores
    bpw = BATCH // nw
    mesh = plsc.VectorSubcoreMesh(core_axis_name="c", subcore_axis_name="s")

    @functools.partial(
        pl.kernel, mesh=mesh,
        out_type=jax.ShapeDtypeStruct((BATCH, EMB), jnp.float32),
        scratch_types=[
            pltpu.VMEM((bpw,), jnp.int32),
            pltpu.VMEM((bpw, EMB), jnp.float32),
            pltpu.SemaphoreType.DMA,
        ],
    )
    def gather(table_hbm, idx_hbm, out_hbm, idx_v, rows_v, sem):
        wid = lax.axis_index("s") * info.num_cores + lax.axis_index("c")
        base = wid * bpw
        pltpu.sync_copy(idx_hbm.at[pl.ds(base, bpw)], idx_v)
        pltpu.async_copy(table_hbm.at[idx_v], rows_v, sem).wait()
        pltpu.sync_copy(rows_v, out_hbm.at[pl.ds(base, bpw)])

    return gather(table, idx)


def kernel(context, device_category, start_time, end_time, host, W):
    cat = device_category.astype(jnp.int32)
    e_actv, e_ap, idx3d = _fused(context, W, cat[:, None])
    e_an = _sc_gather(e_actv, idx3d.reshape(BATCH))
    return (e_actv, e_ap, e_an)


# speedup vs baseline: 1.2967x; 1.0035x over previous
import functools

import jax
import jax.numpy as jnp
from jax import lax
from jax.experimental import pallas as pl
from jax.experimental.pallas import tpu as pltpu
from jax.experimental.pallas import tpu_sc as plsc

ALEN = 40
BATCH = 4096
EMB = 128
NCAT = 64
AUGW = EMB + NCAT + 3
RB = 256
NBLK = BATCH // RB
_BIGI = 1 << 30
_ALPHA = float(2.0 ** 25)


def _fused_body(ctx_ref, w_ref, catc_ref, eact_ref, eap_ref, idx_ref,
                e_sc, aug_sc, aug2_sc):
    i = pl.program_id(0)

    @pl.when(i == 0)
    def _init():
        e = lax.dot_general(ctx_ref[:, :ALEN], w_ref[...],
                            (((1,), (0,)), ((), ())),
                            preferred_element_type=jnp.float32)
        e_sc[...] = e
        lane = lax.broadcasted_iota(jnp.int32, (BATCH, NCAT), 1)
        oh = jnp.where(catc_ref[...] == lane, _ALPHA, 0.0)
        sq = jnp.sum(e * e, axis=1, keepdims=True)
        s1 = sq.astype(jnp.bfloat16).astype(jnp.float32)
        r1 = sq - s1
        s2 = r1.astype(jnp.bfloat16).astype(jnp.float32)
        s3 = r1 - s2
        one = jnp.ones((BATCH, 1), jnp.float32)
        aug_sc[...] = jnp.concatenate(
            [e, oh, s1, s2, s3], axis=1).astype(jnp.bfloat16)
        aug2_sc[...] = jnp.concatenate(
            [-2.0 * e, oh, one, one, one], axis=1).astype(jnp.bfloat16)

    cols = pl.ds(i * RB, RB)
    eact_ref[...] = e_sc[cols, :]
    eap_ref[...] = lax.dot_general(ctx_ref[cols, ALEN:], w_ref[...],
                                   (((1,), (0,)), ((), ())),
                                   preferred_element_type=jnp.float32)

    t = lax.dot_general(aug_sc[...], aug2_sc[cols, :],
                        (((1,), (1,)), ((), ())),
                        preferred_element_type=jnp.float32)
    m = jnp.min(t, axis=0, keepdims=True)
    row = lax.broadcasted_iota(jnp.int32, (BATCH, RB), 0)
    idx = jnp.min(jnp.where(t == m, row, _BIGI), axis=0, keepdims=True)
    idx_ref[...] = idx.reshape(RB)


_fused = pl.pallas_call(
    _fused_body,
    grid=(NBLK,),
    in_specs=[
        pl.BlockSpec((BATCH, 2 * ALEN), lambda i: (0, 0)),
        pl.BlockSpec((ALEN, EMB), lambda i: (0, 0)),
        pl.BlockSpec((BATCH, 1), lambda i: (0, 0)),
    ],
    out_specs=[
        pl.BlockSpec((RB, EMB), lambda i: (i, 0)),
        pl.BlockSpec((RB, EMB), lambda i: (i, 0)),
        pl.BlockSpec((RB,), lambda i: (i,)),
    ],
    out_shape=[
        jax.ShapeDtypeStruct((BATCH, EMB), jnp.float32),
        jax.ShapeDtypeStruct((BATCH, EMB), jnp.float32),
        jax.ShapeDtypeStruct((BATCH,), jnp.int32),
    ],
    scratch_shapes=[
        pltpu.VMEM((BATCH, EMB), jnp.float32),
        pltpu.VMEM((BATCH, AUGW), jnp.bfloat16),
        pltpu.VMEM((BATCH, AUGW), jnp.bfloat16),
    ],
)


def _sc_gather(table, idx):
    info = plsc.get_sparse_core_info()
    nw = info.num_cores * info.num_subcores
    bpw = BATCH // nw
    mesh = plsc.VectorSubcoreMesh(core_axis_name="c", subcore_axis_name="s")

    @functools.partial(
        pl.kernel, mesh=mesh,
        out_type=jax.ShapeDtypeStruct((BATCH, EMB), jnp.float32),
        scratch_types=[
            pltpu.VMEM((bpw,), jnp.int32),
            pltpu.VMEM((bpw, EMB), jnp.float32),
            pltpu.SemaphoreType.DMA,
        ],
    )
    def gather(table_hbm, idx_hbm, out_hbm, idx_v, rows_v, sem):
        wid = lax.axis_index("s") * info.num_cores + lax.axis_index("c")
        base = wid * bpw
        pltpu.sync_copy(idx_hbm.at[pl.ds(base, bpw)], idx_v)
        pltpu.async_copy(table_hbm.at[idx_v], rows_v, sem).wait()
        pltpu.sync_copy(rows_v, out_hbm.at[pl.ds(base, bpw)])

    return gather(table, idx)


def kernel(context, device_category, start_time, end_time, host, W):
    cat = device_category.astype(jnp.int32)
    e_actv, e_ap, idx1d = _fused(context, W, cat[:, None])
    e_an = _sc_gather(e_actv, idx1d)
    return (e_actv, e_ap, e_an)


# speedup vs baseline: 1.4096x; 1.0871x over previous
import functools

import jax
import jax.numpy as jnp
from jax import lax
from jax.experimental import pallas as pl
from jax.experimental.pallas import tpu as pltpu
from jax.experimental.pallas import tpu_sc as plsc

ALEN = 40
BATCH = 4096
EMB = 128
NCAT = 64
AUGW = EMB + NCAT + 3
RB = 512
NBLK = BATCH // RB
_BIGI = 1 << 30
_ALPHA = float(2.0 ** 25)


def _fused_body(ctx_ref, w_ref, catc_ref, eact_ref, eap_ref, idx_ref,
                e_sc, aug_sc, aug2_sc):
    i = pl.program_id(0)

    @pl.when(i == 0)
    def _init():
        e = lax.dot_general(ctx_ref[:, :ALEN], w_ref[...],
                            (((1,), (0,)), ((), ())),
                            preferred_element_type=jnp.float32)
        e_sc[...] = e
        lane = lax.broadcasted_iota(jnp.int32, (BATCH, NCAT), 1)
        oh = jnp.where(catc_ref[...] == lane, _ALPHA, 0.0)
        sq = jnp.sum(e * e, axis=1, keepdims=True)
        s1 = sq.astype(jnp.bfloat16).astype(jnp.float32)
        r1 = sq - s1
        s2 = r1.astype(jnp.bfloat16).astype(jnp.float32)
        s3 = r1 - s2
        one = jnp.ones((BATCH, 1), jnp.float32)
        aug_sc[...] = jnp.concatenate(
            [e, oh, s1, s2, s3], axis=1).astype(jnp.bfloat16)
        aug2_sc[...] = jnp.concatenate(
            [-2.0 * e, oh, one, one, one], axis=1).astype(jnp.bfloat16)

    cols = pl.ds(i * RB, RB)
    eact_ref[...] = e_sc[cols, :]
    eap_ref[...] = lax.dot_general(ctx_ref[cols, ALEN:], w_ref[...],
                                   (((1,), (0,)), ((), ())),
                                   preferred_element_type=jnp.float32)

    t = lax.dot_general(aug_sc[...], aug2_sc[cols, :],
                        (((1,), (1,)), ((), ())),
                        preferred_element_type=jnp.float32)
    m = jnp.min(t, axis=0, keepdims=True)
    row = lax.broadcasted_iota(jnp.int32, (BATCH, RB), 0)
    idx = jnp.min(jnp.where(t == m, row, _BIGI), axis=0, keepdims=True)
    idx_ref[...] = idx.reshape(RB)


_fused = pl.pallas_call(
    _fused_body,
    grid=(NBLK,),
    in_specs=[
        pl.BlockSpec((BATCH, 2 * ALEN), lambda i: (0, 0)),
        pl.BlockSpec((ALEN, EMB), lambda i: (0, 0)),
        pl.BlockSpec((BATCH, 1), lambda i: (0, 0)),
    ],
    out_specs=[
        pl.BlockSpec((RB, EMB), lambda i: (i, 0)),
        pl.BlockSpec((RB, EMB), lambda i: (i, 0)),
        pl.BlockSpec((RB,), lambda i: (i,)),
    ],
    out_shape=[
        jax.ShapeDtypeStruct((BATCH, EMB), jnp.float32),
        jax.ShapeDtypeStruct((BATCH, EMB), jnp.float32),
        jax.ShapeDtypeStruct((BATCH,), jnp.int32),
    ],
    scratch_shapes=[
        pltpu.VMEM((BATCH, EMB), jnp.float32),
        pltpu.VMEM((BATCH, AUGW), jnp.bfloat16),
        pltpu.VMEM((BATCH, AUGW), jnp.bfloat16),
    ],
)


def _sc_gather(table, idx):
    info = plsc.get_sparse_core_info()
    nw = info.num_cores * info.num_subcores
    bpw = BATCH // nw
    mesh = plsc.VectorSubcoreMesh(core_axis_name="c", subcore_axis_name="s")

    @functools.partial(
        pl.kernel, mesh=mesh,
        out_type=jax.ShapeDtypeStruct((BATCH, EMB), jnp.float32),
        scratch_types=[
            pltpu.VMEM((bpw,), jnp.int32),
            pltpu.VMEM((bpw, EMB), jnp.float32),
            pltpu.SemaphoreType.DMA,
        ],
    )
    def gather(table_hbm, idx_hbm, out_hbm, idx_v, rows_v, sem):
        wid = lax.axis_index("s") * info.num_cores + lax.axis_index("c")
        base = wid * bpw
        pltpu.sync_copy(idx_hbm.at[pl.ds(base, bpw)], idx_v)
        pltpu.async_copy(table_hbm.at[idx_v], rows_v, sem).wait()
        pltpu.sync_copy(rows_v, out_hbm.at[pl.ds(base, bpw)])

    return gather(table, idx)


def kernel(context, device_category, start_time, end_time, host, W):
    cat = device_category.astype(jnp.int32)
    e_actv, e_ap, idx1d = _fused(context, W, cat[:, None])
    e_an = _sc_gather(e_actv, idx1d)
    return (e_actv, e_ap, e_an)
